# Initial kernel scaffold; baseline (speedup 1.0000x reference)
import jax
import jax.numpy as jnp
from jax.experimental import pallas as pl


def kernel(h, edge_index, W, b):
    raise NotImplementedError("write your pallas kernel here")


# speedup vs baseline: 3.4780x; 3.4780x over previous
import functools

import jax
import jax.numpy as jnp
from jax import lax
from jax.experimental import pallas as pl
from jax.experimental.pallas import tpu as pltpu
from jax.experimental.pallas import tpu_sc as plsc

N_NODES = 10000
N_EDGES = 320000
DIM = 128

NC = 2
NS = 16
NW = NC * NS

BATCH = 128
EDGES_PER_W = 10240
NB = EDGES_PER_W // BATCH
E_PAD = NW * EDGES_PER_W
OUT_PAD = 10240
STRIPE = OUT_PAD // NS


def _linear(h, W, b):
    def mm(h_ref, w_ref, b_ref, o_ref):
        acc = lax.dot_general(h_ref[...], w_ref[...],
                              (((1,), (1,)), ((), ())),
                              preferred_element_type=jnp.float32)
        o_ref[...] = acc + b_ref[0][None, :]

    b8 = jnp.broadcast_to(b[None, :], (8, DIM))
    return pl.pallas_call(
        mm,
        grid=(10,),
        in_specs=[
            pl.BlockSpec((1000, DIM), lambda i: (i, 0)),
            pl.BlockSpec((DIM, DIM), lambda i: (0, 0)),
            pl.BlockSpec((8, DIM), lambda i: (0, 0)),
        ],
        out_specs=pl.BlockSpec((1000, DIM), lambda i: (i, 0)),
        out_shape=jax.ShapeDtypeStruct((N_NODES, DIM), jnp.float32),
    )(h, W, b8)


def _make_aggregate():
    mesh = plsc.VectorSubcoreMesh(core_axis_name="c", subcore_axis_name="s")

    @functools.partial(
        pl.kernel,
        mesh=mesh,
        out_type=jax.ShapeDtypeStruct((NC, OUT_PAD, DIM), jnp.float32),
        scratch_types=[
            pltpu.VMEM((NB, BATCH), jnp.int32),
            pltpu.VMEM((NB, BATCH), jnp.int32),
            pltpu.VMEM((BATCH, DIM), jnp.float32),
            pltpu.VMEM_SHARED((OUT_PAD, DIM), jnp.float32),
            pltpu.SemaphoreType.DMA,
        ],
    )
    def agg(h2_hbm, src_hbm, dst_hbm, out_hbm, src_v, dst_v, rows_v, acc_sh, sem):
        c = lax.axis_index("c")
        s = lax.axis_index("s")
        wid = s * NC + c

        z16 = jnp.zeros((16,), jnp.float32)

        def zrow(i, _):
            for cc in range(DIM // 16):
                rows_v[i, pl.ds(cc * 16, 16)] = z16
            return _

        lax.fori_loop(0, BATCH, zrow, None)
        row0 = s * STRIPE
        for k in range(STRIPE // BATCH):
            pltpu.sync_copy(rows_v, acc_sh.at[pl.ds(row0 + k * BATCH, BATCH)])
        plsc.subcore_barrier()

        pltpu.sync_copy(src_hbm.at[wid], src_v)
        pltpu.sync_copy(dst_hbm.at[wid], dst_v)

        def body(bi, _):
            pltpu.async_copy(h2_hbm.at[src_v.at[bi]], rows_v, sem).wait()
            pltpu.sync_copy(rows_v, acc_sh.at[dst_v.at[bi]], add=True)
            return _

        lax.fori_loop(0, NB, body, None)
        plsc.subcore_barrier()

        pltpu.sync_copy(acc_sh.at[pl.ds(row0, STRIPE)],
                        out_hbm.at[c, pl.ds(row0, STRIPE)])

    return agg


_aggregate_sc = _make_aggregate()


def _combine(partials):
    def add2(p_ref, o_ref):
        o_ref[...] = p_ref[0] + p_ref[1]

    return pl.pallas_call(
        add2,
        grid=(10,),
        in_specs=[pl.BlockSpec((NC, 1000, DIM), lambda i: (0, i, 0))],
        out_specs=pl.BlockSpec((1000, DIM), lambda i: (i, 0)),
        out_shape=jax.ShapeDtypeStruct((N_NODES, DIM), jnp.float32),
    )(partials)


def kernel(h, edge_index, W, b):
    h2 = _linear(h, W, b)

    dst = edge_index[0].astype(jnp.int32)
    src = edge_index[1].astype(jnp.int32)
    pad = E_PAD - N_EDGES
    src_p = jnp.concatenate([src, jnp.zeros((pad,), jnp.int32)])
    dst_p = jnp.concatenate([dst, jnp.full((pad,), N_NODES, jnp.int32)])
    src_p = src_p.reshape(NW, NB, BATCH)
    dst_p = dst_p.reshape(NW, NB, BATCH)

    partials = _aggregate_sc(h2, src_p, dst_p)
    return _combine(partials)
